# baseline (speedup 1.0000x reference)
import functools

import jax
import jax.numpy as jnp
from jax import lax
from jax.experimental import pallas as pl
from jax.experimental.pallas import tpu as pltpu
from jax.experimental.pallas import tpu_sc as plsc

_C = 128
_NW = 32
_R = 3


def _gather_add(mem, feat, idx):
    n, d = mem.shape
    b = idx.shape[0]
    n_full = b // _C
    last_base = b - _C
    n_chunks = _NW * (-(-(n_full + 1) // _NW))
    per_w = n_chunks // _NW

    idx_pad = jnp.concatenate(
        [idx[:n_full * _C]]
        + [idx[last_base:]] * (n_chunks - n_full)).reshape(_NW, per_w, _C)

    mesh = plsc.VectorSubcoreMesh(core_axis_name="c", subcore_axis_name="s")

    @functools.partial(
        pl.kernel,
        mesh=mesh,
        out_type=jax.ShapeDtypeStruct((b, d), jnp.float32),
        scratch_types=(
            [pltpu.VMEM((per_w, _C), jnp.int32)]
            + [pltpu.VMEM((_C, d), jnp.float32)] * (2 * _R)
            + [pltpu.SemaphoreType.DMA] * (2 * _R)
        ),
    )
    def k(mem_hbm, feat_hbm, idx_hbm, out_hbm, *scratch):
        pidx = scratch[0]
        bm = scratch[1:1 + _R]
        bf = scratch[1 + _R:1 + 2 * _R]
        ga = scratch[1 + 2 * _R:1 + 3 * _R]
        st = scratch[1 + 3 * _R:1 + 4 * _R]

        wid = lax.axis_index("s") * 2 + lax.axis_index("c")
        c0 = wid * per_w

        def store_base(k_):
            return jnp.minimum((c0 + k_) * _C, last_base)

        def issue_gathers(k_):
            j = k_ % _R
            pltpu.async_copy(mem_hbm.at[pidx.at[k_]], bm[j], ga[j])
            pltpu.async_copy(feat_hbm.at[pidx.at[k_]], bf[j], ga[j])

        def wait_gathers(k_):
            j = k_ % _R
            pltpu.make_async_copy(mem_hbm.at[pidx.at[k_]], bm[j], ga[j]).wait()
            pltpu.make_async_copy(feat_hbm.at[pidx.at[k_]], bf[j], ga[j]).wait()

        def wait_store(j):
            pltpu.make_async_copy(bm[j], out_hbm.at[pl.ds(0, _C)], st[j]).wait()

        pltpu.sync_copy(idx_hbm.at[wid], pidx)

        issue_gathers(0)
        issue_gathers(1)

        for k_ in range(per_w):
            j = k_ % _R
            wait_gathers(k_)

            if k_ + 2 < per_w:
                if k_ >= 1:
                    wait_store((k_ + 2) % _R)
                issue_gathers(k_ + 2)

            def row_body(rr, carry):
                for c in range(d // 16):
                    sl = pl.ds(c * 16, 16)
                    bm[j][rr, sl] = bm[j][rr, sl] + bf[j][rr, sl]
                return carry

            lax.fori_loop(0, _C, row_body, 0)

            pltpu.async_copy(bm[j], out_hbm.at[pl.ds(store_base(k_), _C)],
                             st[j])

        for j in range(_R):
            wait_store(j)

    return k(mem, feat, idx_pad)


def kernel(memory, node_features, time_w, time_b, timestamps, source_nodes, n_layers):
    del time_w, time_b, timestamps, n_layers
    return _gather_add(memory, node_features, source_nodes.astype(jnp.int32))


# speedup vs baseline: 1.0146x; 1.0146x over previous
import functools

import jax
import jax.numpy as jnp
from jax import lax
from jax.experimental import pallas as pl
from jax.experimental.pallas import tpu as pltpu
from jax.experimental.pallas import tpu_sc as plsc

_C = 128
_NW = 32
_R = 5


def _gather_add(mem, feat, idx):
    n, d = mem.shape
    b = idx.shape[0]
    n_full = b // _C
    last_base = b - _C
    n_chunks = _NW * (-(-(n_full + 1) // _NW))
    per_w = n_chunks // _NW

    idx_pad = jnp.concatenate(
        [idx[:n_full * _C]]
        + [idx[last_base:]] * (n_chunks - n_full)).reshape(_NW, per_w, _C)

    mesh = plsc.VectorSubcoreMesh(core_axis_name="c", subcore_axis_name="s")

    @functools.partial(
        pl.kernel,
        mesh=mesh,
        out_type=jax.ShapeDtypeStruct((b, d), jnp.float32),
        scratch_types=(
            [pltpu.VMEM((per_w, _C), jnp.int32)]
            + [pltpu.VMEM((_C, d), jnp.float32)] * _R
            + [pltpu.SemaphoreType.DMA] * (2 * _R)
        ),
    )
    def k(mem_hbm, feat_hbm, idx_hbm, out_hbm, *scratch):
        pidx = scratch[0]
        bm = scratch[1:1 + _R]
        ga = scratch[1 + _R:1 + 2 * _R]
        st = scratch[1 + 2 * _R:1 + 3 * _R]

        wid = lax.axis_index("s") * 2 + lax.axis_index("c")
        c0 = wid * per_w

        def store_base(k_):
            return jnp.minimum((c0 + k_) * _C, last_base)

        def issue_a(k_):
            j = k_ % _R
            pltpu.async_copy(mem_hbm.at[pidx.at[k_]], bm[j], ga[j])

        def wait_ga(k_):
            j = k_ % _R
            pltpu.make_async_copy(mem_hbm.at[pidx.at[k_]], bm[j], ga[j]).wait()

        def issue_b(k_):
            j = k_ % _R
            pltpu.async_copy(feat_hbm.at[pidx.at[k_]], bm[j], ga[j], add=True)

        def wait_store(j):
            pltpu.make_async_copy(bm[j], out_hbm.at[pl.ds(0, _C)], st[j]).wait()

        pltpu.sync_copy(idx_hbm.at[wid], pidx)

        for k_ in range(min(4, per_w)):
            issue_a(k_)
        for k_ in range(min(2, per_w)):
            wait_ga(k_)
            issue_b(k_)

        for k_ in range(per_w):
            j = k_ % _R

            wait_ga(k_)
            pltpu.async_copy(bm[j], out_hbm.at[pl.ds(store_base(k_), _C)],
                             st[j])

            if k_ + 2 < per_w:
                wait_ga(k_ + 2)
                issue_b(k_ + 2)

            if k_ + 4 < per_w:
                if k_ >= 1:
                    wait_store((k_ + 4) % _R)
                issue_a(k_ + 4)

        for j in range(min(_R, per_w)):
            wait_store(j)

    return k(mem, feat, idx_pad)


def kernel(memory, node_features, time_w, time_b, timestamps, source_nodes, n_layers):
    del time_w, time_b, timestamps, n_layers
    return _gather_add(memory, node_features, source_nodes.astype(jnp.int32))


# speedup vs baseline: 1.0257x; 1.0109x over previous
import functools

import jax
import jax.numpy as jnp
from jax import lax
from jax.experimental import pallas as pl
from jax.experimental.pallas import tpu as pltpu
from jax.experimental.pallas import tpu_sc as plsc

_C = 128
_NW = 32
_R = 5


def _gather_add(mem, feat, idx):
    n, d = mem.shape
    b = idx.shape[0]
    n_full = b // _C
    last_base = b - _C
    n_chunks = _NW * (-(-(n_full + 1) // _NW))
    per_w = n_chunks // _NW
    assert per_w > 2 * _R, "pipeline prologue/epilogue assume enough chunks"
    ngroups = per_w // _R - 1

    idx_pad = jnp.concatenate(
        [idx[:n_full * _C]]
        + [idx[last_base:]] * (n_chunks - n_full)).reshape(_NW, per_w, _C)

    mesh = plsc.VectorSubcoreMesh(core_axis_name="c", subcore_axis_name="s")

    @functools.partial(
        pl.kernel,
        mesh=mesh,
        out_type=jax.ShapeDtypeStruct((b, d), jnp.float32),
        scratch_types=(
            [pltpu.VMEM((per_w, _C), jnp.int32)]
            + [pltpu.VMEM((_C, d), jnp.float32)] * _R
            + [pltpu.SemaphoreType.DMA] * (2 * _R)
        ),
    )
    def k(mem_hbm, feat_hbm, idx_hbm, out_hbm, *scratch):
        pidx = scratch[0]
        bm = scratch[1:1 + _R]
        ga = scratch[1 + _R:1 + 2 * _R]
        st = scratch[1 + 2 * _R:1 + 3 * _R]

        wid = lax.axis_index("s") * 2 + lax.axis_index("c")
        c0 = wid * per_w

        def store_base(k_):
            return jnp.minimum((c0 + k_) * _C, last_base)

        def issue_a(k_, j):
            pltpu.async_copy(mem_hbm.at[pidx.at[k_]], bm[j], ga[j])

        def wait_ga(k_, j):
            pltpu.make_async_copy(mem_hbm.at[pidx.at[k_]], bm[j], ga[j]).wait()

        def issue_b(k_, j):
            pltpu.async_copy(feat_hbm.at[pidx.at[k_]], bm[j], ga[j], add=True)

        def issue_store(k_, j):
            pltpu.async_copy(bm[j], out_hbm.at[pl.ds(store_base(k_), _C)],
                             st[j])

        def wait_store(j):
            pltpu.make_async_copy(bm[j], out_hbm.at[pl.ds(0, _C)], st[j]).wait()

        pltpu.sync_copy(idx_hbm.at[wid], pidx)

        for k_ in range(4):
            issue_a(k_, k_)
        for k_ in range(2):
            wait_ga(k_, k_)
            issue_b(k_, k_)

        def group_body(g, carry):
            for j in range(_R):
                i = g * _R + j
                j2, j4 = (j + 2) % _R, (j + 4) % _R

                wait_ga(i, j)
                issue_store(i, j)

                wait_ga(i + 2, j2)
                issue_b(i + 2, j2)

                @pl.when(i >= 1)
                def _():
                    wait_store(j4)

                issue_a(i + 4, j4)

            return carry

        lax.fori_loop(0, ngroups, group_body, 0)

        for i in range(per_w - _R, per_w):
            j = i % _R

            wait_ga(i, j)
            issue_store(i, j)

            if i + 2 < per_w:
                j2 = (i + 2) % _R
                wait_ga(i + 2, j2)
                issue_b(i + 2, j2)

            if i + 4 < per_w:
                j4 = (i + 4) % _R
                wait_store(j4)
                issue_a(i + 4, j4)

        for j in range(_R):
            wait_store(j)

    return k(mem, feat, idx_pad)


def kernel(memory, node_features, time_w, time_b, timestamps, source_nodes, n_layers):
    del time_w, time_b, timestamps, n_layers
    return _gather_add(memory, node_features, source_nodes.astype(jnp.int32))


# speedup vs baseline: 1.2374x; 1.2064x over previous
import functools

import jax
import jax.numpy as jnp
from jax import lax
from jax.experimental import pallas as pl
from jax.experimental.pallas import tpu as pltpu
from jax.experimental.pallas import tpu_sc as plsc

_C = 128
_NW = 32
_R = 6


def _gather_add(mem, feat, idx):
    n, d = mem.shape
    b = idx.shape[0]
    n_chunks = -(-b // _C)
    last_base = b - _C
    q, r = divmod(n_chunks, _NW)
    ngroups = -(-(q + 1) // _R)

    mesh = plsc.VectorSubcoreMesh(core_axis_name="c", subcore_axis_name="s")

    @functools.partial(
        pl.kernel,
        mesh=mesh,
        out_type=jax.ShapeDtypeStruct((b, d), jnp.float32),
        scratch_types=(
            [pltpu.VMEM((_C,), jnp.int32)] * _R
            + [pltpu.VMEM((_C, d), jnp.float32)] * _R
            + [pltpu.SemaphoreType.DMA] * (3 * _R)
        ),
    )
    def k(mem_hbm, feat_hbm, idx_hbm, out_hbm, *scratch):
        idxv = scratch[0:_R]
        bm = scratch[_R:2 * _R]
        ga = scratch[2 * _R:3 * _R]
        st = scratch[3 * _R:4 * _R]
        ix = scratch[4 * _R:5 * _R]

        wid = lax.axis_index("s") * 2 + lax.axis_index("c")
        nmine = q + (wid < r)
        s0 = wid * q + jnp.minimum(wid, r)

        def chunk_base(lc):
            return jnp.minimum((s0 + lc) * _C, last_base)

        def issue_idx(lc, j):
            pltpu.async_copy(idx_hbm.at[pl.ds(chunk_base(lc), _C)], idxv[j], ix[j])

        def wait_idx(j):
            pltpu.make_async_copy(idx_hbm.at[pl.ds(0, _C)], idxv[j], ix[j]).wait()

        def issue_a(j):
            pltpu.async_copy(mem_hbm.at[idxv[j]], bm[j], ga[j])

        def wait_ga(j):
            pltpu.make_async_copy(mem_hbm.at[idxv[j]], bm[j], ga[j]).wait()

        def issue_b(j):
            pltpu.async_copy(feat_hbm.at[idxv[j]], bm[j], ga[j], add=True)

        def wait_st(j):
            pltpu.make_async_copy(bm[j], out_hbm.at[pl.ds(0, _C)], st[j]).wait()

        for c in range(_R):
            @pl.when(c < nmine)
            def _():
                issue_idx(c, c)

        for c in range(4):
            @pl.when(c < nmine)
            def _():
                wait_idx(c)
                issue_a(c)

        for c in range(2):
            @pl.when(c < nmine)
            def _():
                wait_ga(c)
                issue_b(c)

        def group_body(g, carry):
            for j in range(_R):
                i = g * _R + j
                j2 = (j + 2) % _R
                j4 = (j + 4) % _R

                @pl.when(i < nmine)
                def _():
                    wait_ga(j)
                    pltpu.async_copy(bm[j], out_hbm.at[pl.ds(chunk_base(i), _C)],
                                     st[j])

                @pl.when(i + 2 < nmine)
                def _():
                    wait_ga(j2)
                    issue_b(j2)

                @pl.when(i + 4 < nmine)
                def _():
                    @pl.when(i >= _R - 4)
                    def _():
                        wait_st(j4)
                    wait_idx(j4)
                    issue_a(j4)

                @pl.when(i + _R < nmine)
                def _():
                    issue_idx(i + _R, j)

            return carry

        lax.fori_loop(0, ngroups, group_body, 0)

        for j in range(_R):
            @pl.when(j < nmine)
            def _():
                wait_st(j)

    return k(mem, feat, idx)


def kernel(memory, node_features, time_w, time_b, timestamps, source_nodes, n_layers):
    del time_w, time_b, timestamps, n_layers
    return _gather_add(memory, node_features, source_nodes.astype(jnp.int32))


# speedup vs baseline: 1.2458x; 1.0068x over previous
import functools

import jax
import jax.numpy as jnp
from jax import lax
from jax.experimental import pallas as pl
from jax.experimental.pallas import tpu as pltpu
from jax.experimental.pallas import tpu_sc as plsc

_C = 128
_NW = 32
_R = 5


def _gather_add(mem, feat, idx):
    n, d = mem.shape
    b = idx.shape[0]
    n_chunks = -(-b // _C)
    last_base = b - _C
    q, r = divmod(n_chunks, _NW)
    ngroups = -(-(q + 1) // _R)

    mesh = plsc.VectorSubcoreMesh(core_axis_name="c", subcore_axis_name="s")

    @functools.partial(
        pl.kernel,
        mesh=mesh,
        out_type=jax.ShapeDtypeStruct((b, d), jnp.float32),
        scratch_types=(
            [pltpu.VMEM((_C,), jnp.int32)] * _R
            + [pltpu.VMEM((_C, d), jnp.float32)] * _R
            + [pltpu.SemaphoreType.DMA] * (3 * _R)
        ),
    )
    def k(mem_hbm, feat_hbm, idx_hbm, out_hbm, *scratch):
        idxv = scratch[0:_R]
        bm = scratch[_R:2 * _R]
        ga = scratch[2 * _R:3 * _R]
        st = scratch[3 * _R:4 * _R]
        ix = scratch[4 * _R:5 * _R]

        wid = lax.axis_index("s") * 2 + lax.axis_index("c")
        nmine = q + (wid < r)
        s0 = wid * q + jnp.minimum(wid, r)

        def chunk_base(lc):
            return jnp.minimum((s0 + lc) * _C, last_base)

        def issue_idx(lc, j):
            pltpu.async_copy(idx_hbm.at[pl.ds(chunk_base(lc), _C)], idxv[j], ix[j])

        def wait_idx(j):
            pltpu.make_async_copy(idx_hbm.at[pl.ds(0, _C)], idxv[j], ix[j]).wait()

        def issue_a(j):
            pltpu.async_copy(mem_hbm.at[idxv[j]], bm[j], ga[j])

        def wait_ga(j):
            pltpu.make_async_copy(mem_hbm.at[idxv[j]], bm[j], ga[j]).wait()

        def issue_b(j):
            pltpu.async_copy(feat_hbm.at[idxv[j]], bm[j], ga[j], add=True)

        def wait_st(j):
            pltpu.make_async_copy(bm[j], out_hbm.at[pl.ds(0, _C)], st[j]).wait()

        for c in range(_R):
            @pl.when(c < nmine)
            def _():
                issue_idx(c, c)

        for c in range(4):
            @pl.when(c < nmine)
            def _():
                wait_idx(c)
                issue_a(c)

        for c in range(2):
            @pl.when(c < nmine)
            def _():
                wait_ga(c)
                issue_b(c)


        def group_body(g, carry):
            for j in range(_R):
                i = g * _R + j
                j2 = (j + 2) % _R
                j4 = (j + 4) % _R

                @pl.when(i < nmine)
                def _():
                    wait_ga(j)
                    pltpu.async_copy(bm[j], out_hbm.at[pl.ds(chunk_base(i), _C)],
                                     st[j])

                @pl.when(i + 2 < nmine)
                def _():
                    wait_ga(j2)
                    issue_b(j2)


                @pl.when(i + 4 < nmine)
                def _():
                    @pl.when(i >= _R - 4)
                    def _():
                        wait_st(j4)
                    wait_idx(j4)
                    issue_a(j4)

                @pl.when(i + _R < nmine)
                def _():
                    issue_idx(i + _R, j)

            return carry

        lax.fori_loop(0, ngroups, group_body, 0)

        for j in range(_R):
            @pl.when(j < nmine)
            def _():
                wait_st(j)

    return k(mem, feat, idx)


def kernel(memory, node_features, time_w, time_b, timestamps, source_nodes, n_layers):
    del time_w, time_b, timestamps, n_layers
    return _gather_add(memory, node_features, source_nodes.astype(jnp.int32))
